# baseline (speedup 1.0000x reference)
import functools

import jax
import jax.numpy as jnp
from jax import lax
from jax.experimental import pallas as pl
from jax.experimental.pallas import tpu as pltpu
from jax.experimental.pallas import tpu_sc as plsc

DIM = 128
N_SC = 27
N_GANA = 12
N_TT = 6
NCOMB = N_SC * N_GANA * N_TT
B, L = 4096, 200
NTOK = B * L

NC, NS = 2, 16
NW = NC * NS
CHUNK = 128
TOK_PER_W = NTOK // NW
CHUNKS_PER_W = TOK_PER_W // CHUNK
IDX_ROWS = NTOK // CHUNK
ROWS_PER_W = IDX_ROWS // NW


def _table_body(sc_ref, gn_ref, tt_ref, w_ref, b_ref, g_ref, beta_ref, out_ref):
    w = w_ref[...]
    nt = (((1,), (1,)), ((), ()))
    a_proj = lax.dot_general(sc_ref[...], w[:, :DIM], nt,
                             preferred_element_type=jnp.float32)
    g_proj = lax.dot_general(gn_ref[...], w[:, DIM:DIM + DIM // 4], nt,
                             preferred_element_type=jnp.float32)
    t_proj = lax.dot_general(tt_ref[...], w[:, DIM + DIM // 4:], nt,
                             preferred_element_type=jnp.float32)

    def expand(proj, n, div, mod):
        r = lax.broadcasted_iota(jnp.int32, (NCOMB, n), 0)
        c = lax.broadcasted_iota(jnp.int32, (NCOMB, n), 1)
        oh = ((r // div) % mod == c).astype(jnp.float32)
        return lax.dot_general(oh, proj, (((1,), (0,)), ((), ())),
                               preferred_element_type=jnp.float32)

    x = (expand(a_proj, N_SC, N_GANA * N_TT, N_SC)
         + expand(g_proj, N_GANA, N_TT, N_GANA)
         + expand(t_proj, N_TT, 1, N_TT)
         + b_ref[...])
    mean = jnp.mean(x, axis=-1, keepdims=True)
    xc = x - mean
    var = jnp.mean(xc * xc, axis=-1, keepdims=True)
    out_ref[...] = xc / jnp.sqrt(var + 1e-5) * g_ref[...] + beta_ref[...]


def _combine_body(a_ref, b_ref, c_ref, out_ref):
    out_ref[...] = a_ref[...] * (N_GANA * N_TT) + b_ref[...] * N_TT + c_ref[...]


NBUF = 5
LA = 2


def _gather_body(table_hbm, idx_hbm, out_hbm, idx_v, rows_v, tbl_sh, gsem, ssem):
    sid = lax.axis_index("s")
    wid = sid * NC + lax.axis_index("c")

    @pl.when(sid == 0)
    def _():
        pltpu.sync_copy(table_hbm, tbl_sh)

    pltpu.sync_copy(idx_hbm.at[pl.ds(wid * ROWS_PER_W, ROWS_PER_W)], idx_v)
    plsc.subcore_barrier()
    base = wid * TOK_PER_W

    def g_copy(c, b):
        return pltpu.make_async_copy(
            tbl_sh.at[idx_v.at[c]], rows_v.at[b], gsem.at[b])

    def s_copy(c, b):
        return pltpu.make_async_copy(
            rows_v.at[b], out_hbm.at[pl.ds(base + c * CHUNK, CHUNK)],
            ssem.at[b])

    def step(c, b, do_swait, do_gissue):
        bg = (b + LA) % NBUF
        if do_swait:
            s_copy(c - (NBUF - LA), bg).wait()
        if do_gissue:
            g_copy(c + LA, bg).start()
        g_copy(c, b).wait()
        s_copy(c, b).start()

    for c in range(LA):
        g_copy(c, c).start()
    for b in range(NBUF):
        step(b, b, do_swait=b >= NBUF - LA, do_gissue=True)

    def group(g, _):
        for b in range(NBUF):
            step(g * NBUF + b, b, do_swait=True, do_gissue=True)
        return 0

    lax.fori_loop(1, CHUNKS_PER_W // NBUF - 1, group, 0)

    for b in range(NBUF):
        c = CHUNKS_PER_W - NBUF + b
        step(c, b, do_swait=True, do_gissue=c + LA < CHUNKS_PER_W)
    for c in range(CHUNKS_PER_W - (NBUF - LA), CHUNKS_PER_W):
        s_copy(c, c % NBUF).wait()


def kernel(semantic_class_ids, gana_ids, token_type_ids, sc_table, gana_table,
           tt_table, fuse_w, fuse_b, ln_gamma, ln_beta):
    table = pl.pallas_call(
        _table_body,
        out_shape=jax.ShapeDtypeStruct((NCOMB, DIM), jnp.float32),
    )(sc_table, gana_table, tt_table, fuse_w,
      fuse_b.reshape(1, DIM), ln_gamma.reshape(1, DIM), ln_beta.reshape(1, DIM))

    a2 = semantic_class_ids.astype(jnp.int32).reshape(IDX_ROWS, CHUNK)
    b2 = gana_ids.astype(jnp.int32).reshape(IDX_ROWS, CHUNK)
    c2 = token_type_ids.astype(jnp.int32).reshape(IDX_ROWS, CHUNK)
    grid = 8
    blk = pl.BlockSpec((IDX_ROWS // grid, CHUNK), lambda i: (i, 0))
    idx = pl.pallas_call(
        _combine_body,
        grid=(grid,),
        in_specs=[blk, blk, blk],
        out_specs=blk,
        out_shape=jax.ShapeDtypeStruct((IDX_ROWS, CHUNK), jnp.int32),
    )(a2, b2, c2)

    mesh = plsc.VectorSubcoreMesh(core_axis_name="c", subcore_axis_name="s")
    gather = pl.kernel(
        _gather_body,
        out_type=jax.ShapeDtypeStruct((NTOK, DIM), jnp.float32),
        mesh=mesh,
        scratch_types=[
            pltpu.VMEM((ROWS_PER_W, CHUNK), jnp.int32),
            pltpu.VMEM((NBUF, CHUNK, DIM), jnp.float32),
            pltpu.VMEM_SHARED((NCOMB, DIM), jnp.float32),
            pltpu.SemaphoreType.DMA((NBUF,)),
            pltpu.SemaphoreType.DMA((NBUF,)),
        ],
    )
    out = gather(table, idx)
    return out.reshape(B, L, DIM)


# speedup vs baseline: 1.0401x; 1.0401x over previous
import functools

import jax
import jax.numpy as jnp
from jax import lax
from jax.experimental import pallas as pl
from jax.experimental.pallas import tpu as pltpu
from jax.experimental.pallas import tpu_sc as plsc

DIM = 128
N_SC = 27
N_GANA = 12
N_TT = 6
NCOMB = N_SC * N_GANA * N_TT
B, L = 4096, 200
NTOK = B * L

NC, NS = 2, 16
NW = NC * NS
CHUNK = 128
TOK_PER_W = NTOK // NW
CHUNKS_PER_W = TOK_PER_W // CHUNK
IDX_ROWS = NTOK // CHUNK
ROWS_PER_W = IDX_ROWS // NW


def _table_body(sc_ref, gn_ref, tt_ref, w_ref, b_ref, g_ref, beta_ref, out_ref):
    w = w_ref[...]
    nt = (((1,), (1,)), ((), ()))
    a_proj = lax.dot_general(sc_ref[...], w[:, :DIM], nt,
                             preferred_element_type=jnp.float32)
    g_proj = lax.dot_general(gn_ref[...], w[:, DIM:DIM + DIM // 4], nt,
                             preferred_element_type=jnp.float32)
    t_proj = lax.dot_general(tt_ref[...], w[:, DIM + DIM // 4:], nt,
                             preferred_element_type=jnp.float32)

    def expand(proj, n, div, mod):
        r = lax.broadcasted_iota(jnp.int32, (NCOMB, n), 0)
        c = lax.broadcasted_iota(jnp.int32, (NCOMB, n), 1)
        oh = ((r // div) % mod == c).astype(jnp.float32)
        return lax.dot_general(oh, proj, (((1,), (0,)), ((), ())),
                               preferred_element_type=jnp.float32)

    x = (expand(a_proj, N_SC, N_GANA * N_TT, N_SC)
         + expand(g_proj, N_GANA, N_TT, N_GANA)
         + expand(t_proj, N_TT, 1, N_TT)
         + b_ref[...])
    mean = jnp.mean(x, axis=-1, keepdims=True)
    xc = x - mean
    var = jnp.mean(xc * xc, axis=-1, keepdims=True)
    out_ref[...] = xc / jnp.sqrt(var + 1e-5) * g_ref[...] + beta_ref[...]


def _tc_body(a_ref, b_ref, c_ref, sc_ref, gn_ref, tt_ref, w_ref, bias_ref,
             g_ref, beta_ref, idx_ref, tbl_ref):
    idx_ref[...] = (a_ref[...] * (N_GANA * N_TT) + b_ref[...] * N_TT
                    + c_ref[...])

    @pl.when(pl.program_id(0) == 0)
    def _():
        _table_body(sc_ref, gn_ref, tt_ref, w_ref, bias_ref, g_ref, beta_ref,
                    tbl_ref)


NBUF = 5
LA = 2


def _gather_body(table_hbm, idx_hbm, out_hbm, idx_v, rows_v, tbl_sh, gsem, ssem):
    sid = lax.axis_index("s")
    wid = sid * NC + lax.axis_index("c")

    @pl.when(sid == 0)
    def _():
        pltpu.sync_copy(table_hbm, tbl_sh)

    pltpu.sync_copy(idx_hbm.at[pl.ds(wid * ROWS_PER_W, ROWS_PER_W)], idx_v)
    plsc.subcore_barrier()
    base = wid * TOK_PER_W

    def g_copy(c, b):
        return pltpu.make_async_copy(
            tbl_sh.at[idx_v.at[c]], rows_v.at[b], gsem.at[b])

    def s_copy(c, b):
        return pltpu.make_async_copy(
            rows_v.at[b], out_hbm.at[pl.ds(base + c * CHUNK, CHUNK)],
            ssem.at[b])

    def step(c, b, do_swait, do_gissue):
        bg = (b + LA) % NBUF
        if do_swait:
            s_copy(c - (NBUF - LA), bg).wait()
        if do_gissue:
            g_copy(c + LA, bg).start()
        g_copy(c, b).wait()
        s_copy(c, b).start()

    for c in range(LA):
        g_copy(c, c).start()
    for b in range(NBUF):
        step(b, b, do_swait=b >= NBUF - LA, do_gissue=True)

    def group(g, _):
        for b in range(NBUF):
            step(g * NBUF + b, b, do_swait=True, do_gissue=True)
        return 0

    lax.fori_loop(1, CHUNKS_PER_W // NBUF - 1, group, 0)

    for b in range(NBUF):
        c = CHUNKS_PER_W - NBUF + b
        step(c, b, do_swait=True, do_gissue=c + LA < CHUNKS_PER_W)
    for c in range(CHUNKS_PER_W - (NBUF - LA), CHUNKS_PER_W):
        s_copy(c, c % NBUF).wait()


def kernel(semantic_class_ids, gana_ids, token_type_ids, sc_table, gana_table,
           tt_table, fuse_w, fuse_b, ln_gamma, ln_beta):
    grid = 8
    in_blk = pl.BlockSpec((B // grid, L), lambda i: (i, 0))
    full = lambda shape: pl.BlockSpec(shape, lambda i: (0,) * len(shape))
    idx, table = pl.pallas_call(
        _tc_body,
        grid=(grid,),
        in_specs=[in_blk, in_blk, in_blk,
                  full((N_SC, DIM)), full((N_GANA, DIM // 4)),
                  full((N_TT, DIM // 2)), full((DIM, DIM + DIM // 4 + DIM // 2)),
                  full((1, DIM)), full((1, DIM)), full((1, DIM))],
        out_specs=[in_blk, full((NCOMB, DIM))],
        out_shape=[jax.ShapeDtypeStruct((B, L), jnp.int32),
                   jax.ShapeDtypeStruct((NCOMB, DIM), jnp.float32)],
    )(semantic_class_ids.astype(jnp.int32), gana_ids.astype(jnp.int32),
      token_type_ids.astype(jnp.int32), sc_table, gana_table, tt_table, fuse_w,
      fuse_b.reshape(1, DIM), ln_gamma.reshape(1, DIM), ln_beta.reshape(1, DIM))
    idx = idx.reshape(IDX_ROWS, CHUNK)

    mesh = plsc.VectorSubcoreMesh(core_axis_name="c", subcore_axis_name="s")
    gather = pl.kernel(
        _gather_body,
        out_type=jax.ShapeDtypeStruct((NTOK, DIM), jnp.float32),
        mesh=mesh,
        scratch_types=[
            pltpu.VMEM((ROWS_PER_W, CHUNK), jnp.int32),
            pltpu.VMEM((NBUF, CHUNK, DIM), jnp.float32),
            pltpu.VMEM_SHARED((NCOMB, DIM), jnp.float32),
            pltpu.SemaphoreType.DMA((NBUF,)),
            pltpu.SemaphoreType.DMA((NBUF,)),
        ],
    )
    out = gather(table, idx)
    return out.reshape(B, L, DIM)


# speedup vs baseline: 1.0411x; 1.0010x over previous
import jax
import jax.numpy as jnp
from jax import lax
from jax.experimental import pallas as pl
from jax.experimental.pallas import tpu as pltpu
from jax.experimental.pallas import tpu_sc as plsc

DIM = 128
N_SC = 27
N_GANA = 12
N_TT = 6
NCOMB = N_SC * N_GANA * N_TT
B, L = 4096, 200
NTOK = B * L

NC, NS = 2, 16
NW = NC * NS
CHUNK = 128
TOK_PER_W = NTOK // NW
CHUNKS_PER_W = TOK_PER_W // CHUNK
IDX_ROWS = NTOK // CHUNK
ROWS_PER_W = IDX_ROWS // NW


def _table_body(sc_ref, gn_ref, tt_ref, w_ref, b_ref, g_ref, beta_ref, out_ref):
    w = w_ref[...]
    nt = (((1,), (1,)), ((), ()))
    a_proj = lax.dot_general(sc_ref[...], w[:, :DIM], nt,
                             preferred_element_type=jnp.float32)
    g_proj = lax.dot_general(gn_ref[...], w[:, DIM:DIM + DIM // 4], nt,
                             preferred_element_type=jnp.float32)
    t_proj = lax.dot_general(tt_ref[...], w[:, DIM + DIM // 4:], nt,
                             preferred_element_type=jnp.float32)

    def expand(proj, n, div, mod):
        r = lax.broadcasted_iota(jnp.int32, (NCOMB, n), 0)
        c = lax.broadcasted_iota(jnp.int32, (NCOMB, n), 1)
        oh = ((r // div) % mod == c).astype(jnp.float32)
        return lax.dot_general(oh, proj, (((1,), (0,)), ((), ())),
                               preferred_element_type=jnp.float32)

    x = (expand(a_proj, N_SC, N_GANA * N_TT, N_SC)
         + expand(g_proj, N_GANA, N_TT, N_GANA)
         + expand(t_proj, N_TT, 1, N_TT)
         + b_ref[...])
    mean = jnp.mean(x, axis=-1, keepdims=True)
    xc = x - mean
    var = jnp.mean(xc * xc, axis=-1, keepdims=True)
    out_ref[...] = xc / jnp.sqrt(var + 1e-5) * g_ref[...] + beta_ref[...]


def _tc_body(a_ref, b_ref, c_ref, sc_ref, gn_ref, tt_ref, w_ref, bias_ref,
             g_ref, beta_ref, idx_ref, tbl_ref):
    idx_ref[...] = (a_ref[...] * (N_GANA * N_TT) + b_ref[...] * N_TT
                    + c_ref[...])

    @pl.when(pl.program_id(0) == 0)
    def _():
        _table_body(sc_ref, gn_ref, tt_ref, w_ref, bias_ref, g_ref, beta_ref,
                    tbl_ref)


NBUF = 5
LA = 2


def _gather_body(table_hbm, idx_hbm, out_hbm, idx_v, rows_v, tbl_sh, gsem, ssem):
    sid = lax.axis_index("s")
    wid = sid * NC + lax.axis_index("c")

    @pl.when(sid == 0)
    def _():
        pltpu.sync_copy(table_hbm, tbl_sh)

    pltpu.sync_copy(idx_hbm.at[pl.ds(wid * ROWS_PER_W, ROWS_PER_W)], idx_v)
    plsc.subcore_barrier()
    base = wid * TOK_PER_W

    def g_copy(c, b):
        return pltpu.make_async_copy(
            tbl_sh.at[idx_v.at[c]], rows_v.at[b], gsem.at[b])

    def s_copy(c, b):
        return pltpu.make_async_copy(
            rows_v.at[b], out_hbm.at[pl.ds(base + c * CHUNK, CHUNK)],
            ssem.at[b])

    def step(c, b, do_swait, do_gissue):
        bg = (b + LA) % NBUF
        if do_swait:
            s_copy(c - (NBUF - LA), bg).wait()
        if do_gissue:
            g_copy(c + LA, bg).start()
        g_copy(c, b).wait()
        s_copy(c, b).start()

    for c in range(LA):
        g_copy(c, c).start()
    for b in range(NBUF):
        step(b, b, do_swait=b >= NBUF - LA, do_gissue=True)

    def group(g, _):
        for b in range(NBUF):
            step(g * NBUF + b, b, do_swait=True, do_gissue=True)
        return 0

    lax.fori_loop(1, CHUNKS_PER_W // NBUF - 1, group, 0)

    for b in range(NBUF):
        c = CHUNKS_PER_W - NBUF + b
        step(c, b, do_swait=True, do_gissue=c + LA < CHUNKS_PER_W)
    for c in range(CHUNKS_PER_W - (NBUF - LA), CHUNKS_PER_W):
        s_copy(c, c % NBUF).wait()


def kernel(semantic_class_ids, gana_ids, token_type_ids, sc_table, gana_table,
           tt_table, fuse_w, fuse_b, ln_gamma, ln_beta):
    grid = 8
    in_blk = pl.BlockSpec((B // grid, L), lambda i: (i, 0))
    full = lambda shape: pl.BlockSpec(shape, lambda i: (0,) * len(shape))
    idx, table = pl.pallas_call(
        _tc_body,
        grid=(grid,),
        in_specs=[in_blk, in_blk, in_blk,
                  full((N_SC, DIM)), full((N_GANA, DIM // 4)),
                  full((N_TT, DIM // 2)), full((DIM, DIM + DIM // 4 + DIM // 2)),
                  full((1, DIM)), full((1, DIM)), full((1, DIM))],
        out_specs=[in_blk, full((NCOMB, DIM))],
        out_shape=[jax.ShapeDtypeStruct((B, L), jnp.int32),
                   jax.ShapeDtypeStruct((NCOMB, DIM), jnp.float32)],
    )(semantic_class_ids.astype(jnp.int32), gana_ids.astype(jnp.int32),
      token_type_ids.astype(jnp.int32), sc_table, gana_table, tt_table, fuse_w,
      fuse_b.reshape(1, DIM), ln_gamma.reshape(1, DIM), ln_beta.reshape(1, DIM))
    idx = idx.reshape(IDX_ROWS, CHUNK)

    mesh = plsc.VectorSubcoreMesh(core_axis_name="c", subcore_axis_name="s")
    gather = pl.kernel(
        _gather_body,
        out_type=jax.ShapeDtypeStruct((NTOK, DIM), jnp.float32),
        mesh=mesh,
        scratch_types=[
            pltpu.VMEM((ROWS_PER_W, CHUNK), jnp.int32),
            pltpu.VMEM((NBUF, CHUNK, DIM), jnp.float32),
            pltpu.VMEM_SHARED((NCOMB, DIM), jnp.float32),
            pltpu.SemaphoreType.DMA((NBUF,)),
            pltpu.SemaphoreType.DMA((NBUF,)),
        ],
    )
    out = gather(table, idx)
    return out.reshape(B, L, DIM)
